# baseline (speedup 1.0000x reference)
import functools

import jax
import jax.numpy as jnp
from jax import lax
from jax.experimental import pallas as pl
from jax.experimental.pallas import tpu as pltpu
from jax.experimental.pallas import tpu_sc as plsc

_INTERPRET = False

N_NODES = 10000
D_X = 128
D_H = 64

_NW = 32
_BS = 320
_NPAD = _NW * _BS
_EW = 8000
_CH = 128


def _scatter_max_sc(h, dst, selfinit_pad):
    n_win = dst.shape[0] // _EW
    mesh = plsc.VectorSubcoreMesh(core_axis_name="c", subcore_axis_name="s",
                                  num_cores=2, num_subcores=16)

    @functools.partial(
        pl.kernel,
        out_type=jax.ShapeDtypeStruct((_NPAD, D_H), jnp.float32),
        mesh=mesh,
        compiler_params=pltpu.CompilerParams(needs_layout_passes=False),
        scratch_types=[
            pltpu.VMEM((_BS, D_H), jnp.float32),
            pltpu.VMEM((_EW,), jnp.int32),
            pltpu.VMEM((_EW // _CH + 2, _CH), jnp.int32),
            pltpu.VMEM((_EW + 192,), jnp.int32),
            pltpu.VMEM((_CH, 2 * D_H), jnp.float32),
            pltpu.SemaphoreType.DMA,
        ],
    )
    def body(h_hbm, dst_hbm, self_hbm, out_hbm,
             agg, wdst, idbuf, nodbuf, rowbuf, sem):
        wid = lax.axis_index("s") * 2 + lax.axis_index("c")
        lo = wid * _BS
        pltpu.sync_copy(self_hbm.at[pl.ds(lo, _BS)], agg)
        iota = lax.iota(jnp.int32, 16)
        ftrue = iota < 16
        zeros16 = jnp.zeros((16,), jnp.int32)

        def window(win, _):
            wbase = win * _EW
            pltpu.sync_copy(dst_hbm.at[pl.ds(wbase, _EW)], wdst)

            def scan(v, cnt):
                d16 = wdst[pl.ds(v * 16, 16)]
                rel = d16 - lo
                m = (rel >= 0) & (rel < _BS)
                ids = wbase + v * 16 + iota
                csum = jnp.cumsum(jnp.where(m, jnp.int32(1), jnp.int32(0)))
                pos = cnt - 1 + csum
                nodpar = rel | ((ids & 1) << 16)
                plsc.store_scatter(idbuf, [pos >> 7, pos & (_CH - 1)],
                                   ids >> 1, mask=m)
                plsc.store_scatter(nodbuf, [pos], nodpar, mask=m)
                return cnt + csum[15]

            cnt = lax.fori_loop(0, _EW // 16, scan, jnp.int32(0))

            def padz(k, _):
                pz = cnt + k * 16 + iota
                plsc.store_scatter(idbuf, [pz >> 7, pz & (_CH - 1)], zeros16,
                                   mask=ftrue)
                return 0
            lax.fori_loop(0, _CH // 16, padz, 0)

            def chunk(c, _):
                base = c * _CH
                pltpu.async_copy(h_hbm.at[idbuf.at[c]], rowbuf, sem).wait()
                me = jnp.minimum(_CH, cnt - base)

                def rmw(e, _):
                    nodpar = nodbuf[pl.ds(base + e, 16)][0]
                    node = nodpar & 0xFFFF
                    par = nodpar >> 16
                    for j in range(4):
                        sl = pl.ds(j * 16, 16)
                        hsl = pl.ds(par * D_H + j * 16, 16)
                        agg[node, sl] = jnp.maximum(agg[node, sl],
                                                    rowbuf[e, hsl])
                    return 0
                lax.fori_loop(0, me, rmw, 0)
                return 0

            nch = (cnt + _CH - 1) // _CH
            lax.fori_loop(0, nch, chunk, 0)
            return 0

        lax.fori_loop(0, n_win, window, 0)
        pltpu.sync_copy(agg, out_hbm.at[pl.ds(lo, _BS)])

    return body(h, dst, selfinit_pad)


_GC = 80


def _gather_sc(p, src, dst):
    e_total = src.shape[0]
    per_tile = e_total // _NW
    n_chunks = per_tile // _GC
    mesh = plsc.VectorSubcoreMesh(core_axis_name="c", subcore_axis_name="s",
                                  num_cores=2, num_subcores=16)

    @functools.partial(
        pl.kernel,
        out_type=jax.ShapeDtypeStruct((e_total // 2, 2 * D_H), jnp.float32),
        mesh=mesh,
        compiler_params=pltpu.CompilerParams(needs_layout_passes=False),
        scratch_types=[
            pltpu.VMEM_SHARED((N_NODES, 2 * D_H), jnp.float32),
            pltpu.VMEM((per_tile,), jnp.int32),
            pltpu.VMEM((per_tile,), jnp.int32),
            pltpu.VMEM((_GC, 2 * D_H), jnp.float32),
            pltpu.VMEM((_GC, 2 * D_H), jnp.float32),
            pltpu.VMEM((_GC // 2, 2 * D_H), jnp.float32),
            pltpu.SemaphoreType.DMA,
            pltpu.SemaphoreType.DMA,
        ],
    )
    def body(p_hbm, src_hbm, dst_hbm, t2_hbm,
             psp, srcw, dstw, abuf, bbuf, obuf, sema, semb):
        cid = lax.axis_index("c")
        sid = lax.axis_index("s")
        wid = sid * 2 + cid
        tb = wid * per_tile
        tb2 = wid * (per_tile // 2)

        @pl.when(sid == 0)
        def _stage():
            pltpu.sync_copy(p_hbm, psp)
        plsc.subcore_barrier()

        pltpu.sync_copy(src_hbm.at[pl.ds(tb, per_tile)], srcw)
        pltpu.sync_copy(dst_hbm.at[pl.ds(tb, per_tile)], dstw)

        def chunk(c, _):
            cb = c * _GC
            ca = pltpu.async_copy(psp.at[srcw.at[pl.ds(cb, _GC)]], abuf, sema)
            cbm = pltpu.async_copy(psp.at[dstw.at[pl.ds(cb, _GC)]], bbuf, semb)
            ca.wait()
            cbm.wait()

            def pair(i, _):
                for j in range(4):
                    slo = pl.ds(j * 16, 16)
                    shi = pl.ds(D_H + j * 16, 16)
                    obuf[i, slo] = abuf[2 * i, slo] - bbuf[2 * i, shi]
                    obuf[i, shi] = abuf[2 * i + 1, slo] - bbuf[2 * i + 1, shi]
                return 0
            lax.fori_loop(0, _GC // 2, pair, 0)
            pltpu.sync_copy(
                obuf, t2_hbm.at[pl.ds(tb2 + c * (_GC // 2), _GC // 2)])
            return 0

        lax.fori_loop(0, n_chunks, chunk, 0)

    return body(p, src, dst)


def _node_pre_body(x_ref, pos_ref, w1xt_ref, w1pt_ref, b1_ref, w2t_ref,
                   u_ref, w_ref, self_ref):
    xb = x_ref[...]
    pb = pos_ref[...]
    w_blk = jnp.dot(pb, w1pt_ref[...], preferred_element_type=jnp.float32)
    ux = jnp.dot(xb, w1xt_ref[...], preferred_element_type=jnp.float32)
    u_blk = ux + w_blk + b1_ref[...]
    u_ref[...] = u_blk
    w_ref[...] = w_blk
    self_ref[...] = jnp.dot(jax.nn.relu(ux + b1_ref[...]), w2t_ref[...],
                            preferred_element_type=jnp.float32)


def _node_pre(x, pos, w1xt, w1pt, b1, w2t, bn=1000):
    n = x.shape[0]
    grid = (n // bn,)
    return pl.pallas_call(
        _node_pre_body,
        grid=grid,
        in_specs=[
            pl.BlockSpec((bn, D_X), lambda i: (i, 0)),
            pl.BlockSpec((bn, 3), lambda i: (i, 0)),
            pl.BlockSpec((D_X, D_H), lambda i: (0, 0)),
            pl.BlockSpec((3, D_H), lambda i: (0, 0)),
            pl.BlockSpec((1, D_H), lambda i: (0, 0)),
            pl.BlockSpec((D_H, D_H), lambda i: (0, 0)),
        ],
        out_specs=[
            pl.BlockSpec((bn, D_H), lambda i: (i, 0)),
            pl.BlockSpec((bn, D_H), lambda i: (i, 0)),
            pl.BlockSpec((bn, D_H), lambda i: (i, 0)),
        ],
        out_shape=[
            jax.ShapeDtypeStruct((n, D_H), jnp.float32),
            jax.ShapeDtypeStruct((n, D_H), jnp.float32),
            jax.ShapeDtypeStruct((n, D_H), jnp.float32),
        ],
        interpret=_INTERPRET,
    )(x, pos, w1xt, w1pt, b1.reshape(1, D_H), w2t)


def _edge_mlp_body(t_ref, w2t_ref, h_ref):
    h_ref[...] = jnp.dot(jax.nn.relu(t_ref[...]), w2t_ref[...],
                         preferred_element_type=jnp.float32)


def _edge_mlp2(t2, w2bd, be=1000):
    e2 = t2.shape[0]
    grid = (e2 // be,)
    return pl.pallas_call(
        _edge_mlp_body,
        grid=grid,
        in_specs=[
            pl.BlockSpec((be, 2 * D_H), lambda i: (i, 0)),
            pl.BlockSpec((2 * D_H, 2 * D_H), lambda i: (0, 0)),
        ],
        out_specs=pl.BlockSpec((be, 2 * D_H), lambda i: (i, 0)),
        out_shape=jax.ShapeDtypeStruct((e2, 2 * D_H), jnp.float32),
        interpret=_INTERPRET,
    )(t2, w2bd)


def _global_mlp_body(a_ref, b2_ref, g1t_ref, g1_ref, g2t_ref, g2_ref,
                     g3t_ref, g3_ref, o_ref):
    a = a_ref[...] + b2_ref[...]
    a = jax.nn.relu(jnp.dot(a, g1t_ref[...], preferred_element_type=jnp.float32)
                    + g1_ref[...])
    a = jax.nn.relu(jnp.dot(a, g2t_ref[...], preferred_element_type=jnp.float32)
                    + g2_ref[...])
    o_ref[...] = jnp.dot(a, g3t_ref[...], preferred_element_type=jnp.float32) \
        + g3_ref[...]


def _global_mlp(agg, b2, g1t, g1, g2t, g2, g3t, g3, bn=1000):
    n = agg.shape[0]
    grid = (n // bn,)
    return pl.pallas_call(
        _global_mlp_body,
        grid=grid,
        in_specs=[
            pl.BlockSpec((bn, D_H), lambda i: (i, 0)),
            pl.BlockSpec((1, D_H), lambda i: (0, 0)),
            pl.BlockSpec((D_H, 32), lambda i: (0, 0)),
            pl.BlockSpec((1, 32), lambda i: (0, 0)),
            pl.BlockSpec((32, 128), lambda i: (0, 0)),
            pl.BlockSpec((1, 128), lambda i: (0, 0)),
            pl.BlockSpec((128, 128), lambda i: (0, 0)),
            pl.BlockSpec((1, 128), lambda i: (0, 0)),
        ],
        out_specs=pl.BlockSpec((bn, 128), lambda i: (i, 0)),
        out_shape=jax.ShapeDtypeStruct((n, 128), jnp.float32),
        interpret=_INTERPRET,
    )(agg, b2.reshape(1, D_H), g1t, g1.reshape(1, 32), g2t, g2.reshape(1, 128),
      g3t, g3.reshape(1, 128))


def kernel(x, pos, edge_index, W1, b1, W2, b2, G1, g1, G2, g2, G3, g3):
    src = edge_index[0].astype(jnp.int32)
    dst = edge_index[1].astype(jnp.int32)
    w1xt = W1[:, :D_X].T
    w1pt = W1[:, D_X:].T
    u, w, selfinit = _node_pre(x, pos, w1xt, w1pt, b1, W2.T)
    e_total = src.shape[0]
    t2 = (u[src] - w[dst]).reshape(e_total // 2, 2 * D_H)
    w2t = W2.T
    w2bd = jnp.zeros((2 * D_H, 2 * D_H), jnp.float32)
    w2bd = w2bd.at[:D_H, :D_H].set(w2t).at[D_H:, D_H:].set(w2t)
    h2 = _edge_mlp2(t2, w2bd)
    selfpad = jnp.concatenate(
        [selfinit, jnp.zeros((_NPAD - N_NODES, D_H), jnp.float32)])
    agg = _scatter_max_sc(h2, dst, selfpad)[:N_NODES]
    return _global_mlp(agg, b2, G1.T, g1, G2.T, g2, G3.T, g3)


# speedup vs baseline: 1.8980x; 1.8980x over previous
import functools

import jax
import jax.numpy as jnp
from jax import lax
from jax.experimental import pallas as pl
from jax.experimental.pallas import tpu as pltpu
from jax.experimental.pallas import tpu_sc as plsc

_INTERPRET = False

N_NODES = 10000
D_X = 128
D_H = 64

_NW = 32
_BS = 320
_NPAD = _NW * _BS
_EW = 8000
_CH = 32


_NRING = 8
_SHIFT = 5
_UNROLL = 4


def _scatter_max_sc(h, dst, selfinit_pad):
    n_win = dst.shape[0] // _EW
    mesh = plsc.VectorSubcoreMesh(core_axis_name="c", subcore_axis_name="s",
                                  num_cores=2, num_subcores=16)

    @functools.partial(
        pl.kernel,
        out_type=jax.ShapeDtypeStruct((_NPAD, D_H), jnp.float32),
        mesh=mesh,
        compiler_params=pltpu.CompilerParams(needs_layout_passes=False),
        scratch_types=[
            pltpu.VMEM((_BS, D_H), jnp.float32),
            pltpu.VMEM((_EW,), jnp.int32),
            pltpu.VMEM((_EW // _CH + 2, _CH), jnp.int32),
            pltpu.VMEM((_EW + 192,), jnp.int32),
            [pltpu.VMEM((_CH, 2 * D_H), jnp.float32) for _ in range(_NRING)],
            [pltpu.SemaphoreType.DMA for _ in range(_NRING)],
        ],
    )
    def body(h_hbm, dst_hbm, self_hbm, out_hbm,
             agg, wdst, idbuf, nodbuf, rowbufs, sems):
        wid = lax.axis_index("s") * 2 + lax.axis_index("c")
        lo = wid * _BS
        pltpu.sync_copy(self_hbm.at[pl.ds(lo, _BS)], agg)
        iota = lax.iota(jnp.int32, 16)
        ftrue = iota < 16
        zeros16 = jnp.zeros((16,), jnp.int32)

        def window(win, _):
            wbase = win * _EW
            pltpu.sync_copy(dst_hbm.at[pl.ds(wbase, _EW)], wdst)

            def scan(v, cnt):
                for uu in range(_UNROLL):
                    vv = v * _UNROLL + uu
                    d16 = wdst[pl.ds(vv * 16, 16)]
                    rel = d16 - lo
                    m = (rel >= 0) & (rel < _BS)
                    ids = wbase + vv * 16 + iota
                    csum = jnp.cumsum(jnp.where(m, jnp.int32(1), jnp.int32(0)))
                    pos = cnt - 1 + csum
                    nodpar = rel | ((ids & 1) << 16)
                    plsc.store_scatter(idbuf, [pos >> _SHIFT,
                                               pos & (_CH - 1)],
                                       ids >> 1, mask=m)
                    plsc.store_scatter(nodbuf, [pos], nodpar, mask=m)
                    cnt = cnt + csum[15]
                return cnt

            cnt = lax.fori_loop(0, _EW // 16 // _UNROLL, scan, jnp.int32(0))

            def padz(k, _):
                pz = cnt + k * 16 + iota
                plsc.store_scatter(idbuf, [pz >> _SHIFT, pz & (_CH - 1)],
                                   zeros16, mask=ftrue)
                return 0
            lax.fori_loop(0, max(_CH // 16, 1), padz, 0)

            nch = (cnt + _CH - 1) >> _SHIFT

            for b in range(_NRING):
                @pl.when(b < nch)
                def _issue0(b=b):
                    pltpu.async_copy(h_hbm.at[idbuf.at[b]], rowbufs[b],
                                     sems[b])

            def group(g, _):
                for b in range(_NRING):
                    c = g * _NRING + b

                    @pl.when(c < nch)
                    def _do(b=b, c=c):
                        pltpu.make_async_copy(h_hbm.at[idbuf.at[c]],
                                              rowbufs[b], sems[b]).wait()
                        me = jnp.minimum(_CH, cnt - c * _CH)

                        def rmw(e, _):
                            nodpar = nodbuf[pl.ds(c * _CH + e, 16)][0]
                            node = nodpar & 0xFFFF
                            par = nodpar >> 16
                            for j in range(4):
                                sl = pl.ds(j * 16, 16)
                                hsl = pl.ds(par * D_H + j * 16, 16)
                                agg[node, sl] = jnp.maximum(
                                    agg[node, sl], rowbufs[b][e, hsl])
                            return 0
                        lax.fori_loop(0, me, rmw, 0)

                        @pl.when(c + _NRING < nch)
                        def _issue():
                            pltpu.async_copy(
                                h_hbm.at[idbuf.at[c + _NRING]],
                                rowbufs[b], sems[b])
                return 0

            lax.fori_loop(0, (nch + _NRING - 1) // _NRING, group, 0)
            return 0

        lax.fori_loop(0, n_win, window, 0)
        pltpu.sync_copy(agg, out_hbm.at[pl.ds(lo, _BS)])

    return body(h, dst, selfinit_pad)


_GC = 80


def _gather_sc(p, src, dst):
    e_total = src.shape[0]
    per_tile = e_total // _NW
    n_chunks = per_tile // _GC
    mesh = plsc.VectorSubcoreMesh(core_axis_name="c", subcore_axis_name="s",
                                  num_cores=2, num_subcores=16)

    @functools.partial(
        pl.kernel,
        out_type=jax.ShapeDtypeStruct((e_total // 2, 2 * D_H), jnp.float32),
        mesh=mesh,
        compiler_params=pltpu.CompilerParams(needs_layout_passes=False),
        scratch_types=[
            pltpu.VMEM_SHARED((N_NODES, 2 * D_H), jnp.float32),
            pltpu.VMEM((per_tile,), jnp.int32),
            pltpu.VMEM((per_tile,), jnp.int32),
            pltpu.VMEM((_GC, 2 * D_H), jnp.float32),
            pltpu.VMEM((_GC, 2 * D_H), jnp.float32),
            pltpu.VMEM((_GC // 2, 2 * D_H), jnp.float32),
            pltpu.SemaphoreType.DMA,
            pltpu.SemaphoreType.DMA,
        ],
    )
    def body(p_hbm, src_hbm, dst_hbm, t2_hbm,
             psp, srcw, dstw, abuf, bbuf, obuf, sema, semb):
        cid = lax.axis_index("c")
        sid = lax.axis_index("s")
        wid = sid * 2 + cid
        tb = wid * per_tile
        tb2 = wid * (per_tile // 2)

        @pl.when(sid == 0)
        def _stage():
            pltpu.sync_copy(p_hbm, psp)
        plsc.subcore_barrier()

        pltpu.sync_copy(src_hbm.at[pl.ds(tb, per_tile)], srcw)
        pltpu.sync_copy(dst_hbm.at[pl.ds(tb, per_tile)], dstw)

        def chunk(c, _):
            cb = c * _GC
            ca = pltpu.async_copy(psp.at[srcw.at[pl.ds(cb, _GC)]], abuf, sema)
            cbm = pltpu.async_copy(psp.at[dstw.at[pl.ds(cb, _GC)]], bbuf, semb)
            ca.wait()
            cbm.wait()

            def pair(i, _):
                for j in range(4):
                    slo = pl.ds(j * 16, 16)
                    shi = pl.ds(D_H + j * 16, 16)
                    obuf[i, slo] = abuf[2 * i, slo] - bbuf[2 * i, shi]
                    obuf[i, shi] = abuf[2 * i + 1, slo] - bbuf[2 * i + 1, shi]
                return 0
            lax.fori_loop(0, _GC // 2, pair, 0)
            pltpu.sync_copy(
                obuf, t2_hbm.at[pl.ds(tb2 + c * (_GC // 2), _GC // 2)])
            return 0

        lax.fori_loop(0, n_chunks, chunk, 0)

    return body(p, src, dst)


def _node_pre_body(x_ref, pos_ref, w1xt_ref, w1pt_ref, b1_ref, w2t_ref,
                   u_ref, w_ref, self_ref):
    xb = x_ref[...]
    pb = pos_ref[...]
    w_blk = jnp.dot(pb, w1pt_ref[...], preferred_element_type=jnp.float32)
    ux = jnp.dot(xb, w1xt_ref[...], preferred_element_type=jnp.float32)
    u_blk = ux + w_blk + b1_ref[...]
    u_ref[...] = u_blk
    w_ref[...] = w_blk
    self_ref[...] = jnp.dot(jax.nn.relu(ux + b1_ref[...]), w2t_ref[...],
                            preferred_element_type=jnp.float32)


def _node_pre(x, pos, w1xt, w1pt, b1, w2t, bn=1000):
    n = x.shape[0]
    grid = (n // bn,)
    return pl.pallas_call(
        _node_pre_body,
        grid=grid,
        in_specs=[
            pl.BlockSpec((bn, D_X), lambda i: (i, 0)),
            pl.BlockSpec((bn, 3), lambda i: (i, 0)),
            pl.BlockSpec((D_X, D_H), lambda i: (0, 0)),
            pl.BlockSpec((3, D_H), lambda i: (0, 0)),
            pl.BlockSpec((1, D_H), lambda i: (0, 0)),
            pl.BlockSpec((D_H, D_H), lambda i: (0, 0)),
        ],
        out_specs=[
            pl.BlockSpec((bn, D_H), lambda i: (i, 0)),
            pl.BlockSpec((bn, D_H), lambda i: (i, 0)),
            pl.BlockSpec((bn, D_H), lambda i: (i, 0)),
        ],
        out_shape=[
            jax.ShapeDtypeStruct((n, D_H), jnp.float32),
            jax.ShapeDtypeStruct((n, D_H), jnp.float32),
            jax.ShapeDtypeStruct((n, D_H), jnp.float32),
        ],
        interpret=_INTERPRET,
    )(x, pos, w1xt, w1pt, b1.reshape(1, D_H), w2t)


def _edge_mlp_body(t_ref, w2t_ref, h_ref):
    h_ref[...] = jnp.dot(jax.nn.relu(t_ref[...]), w2t_ref[...],
                         preferred_element_type=jnp.float32)


def _edge_mlp2(t2, w2bd, be=1000):
    e2 = t2.shape[0]
    grid = (e2 // be,)
    return pl.pallas_call(
        _edge_mlp_body,
        grid=grid,
        in_specs=[
            pl.BlockSpec((be, 2 * D_H), lambda i: (i, 0)),
            pl.BlockSpec((2 * D_H, 2 * D_H), lambda i: (0, 0)),
        ],
        out_specs=pl.BlockSpec((be, 2 * D_H), lambda i: (i, 0)),
        out_shape=jax.ShapeDtypeStruct((e2, 2 * D_H), jnp.float32),
        interpret=_INTERPRET,
    )(t2, w2bd)


def _global_mlp_body(a_ref, b2_ref, g1t_ref, g1_ref, g2t_ref, g2_ref,
                     g3t_ref, g3_ref, o_ref):
    a = a_ref[...] + b2_ref[...]
    a = jax.nn.relu(jnp.dot(a, g1t_ref[...], preferred_element_type=jnp.float32)
                    + g1_ref[...])
    a = jax.nn.relu(jnp.dot(a, g2t_ref[...], preferred_element_type=jnp.float32)
                    + g2_ref[...])
    o_ref[...] = jnp.dot(a, g3t_ref[...], preferred_element_type=jnp.float32) \
        + g3_ref[...]


def _global_mlp(agg, b2, g1t, g1, g2t, g2, g3t, g3, bn=1000):
    n = agg.shape[0]
    grid = (n // bn,)
    return pl.pallas_call(
        _global_mlp_body,
        grid=grid,
        in_specs=[
            pl.BlockSpec((bn, D_H), lambda i: (i, 0)),
            pl.BlockSpec((1, D_H), lambda i: (0, 0)),
            pl.BlockSpec((D_H, 32), lambda i: (0, 0)),
            pl.BlockSpec((1, 32), lambda i: (0, 0)),
            pl.BlockSpec((32, 128), lambda i: (0, 0)),
            pl.BlockSpec((1, 128), lambda i: (0, 0)),
            pl.BlockSpec((128, 128), lambda i: (0, 0)),
            pl.BlockSpec((1, 128), lambda i: (0, 0)),
        ],
        out_specs=pl.BlockSpec((bn, 128), lambda i: (i, 0)),
        out_shape=jax.ShapeDtypeStruct((n, 128), jnp.float32),
        interpret=_INTERPRET,
    )(agg, b2.reshape(1, D_H), g1t, g1.reshape(1, 32), g2t, g2.reshape(1, 128),
      g3t, g3.reshape(1, 128))


def kernel(x, pos, edge_index, W1, b1, W2, b2, G1, g1, G2, g2, G3, g3):
    src = edge_index[0].astype(jnp.int32)
    dst = edge_index[1].astype(jnp.int32)
    w1xt = W1[:, :D_X].T
    w1pt = W1[:, D_X:].T
    u, w, selfinit = _node_pre(x, pos, w1xt, w1pt, b1, W2.T)
    e_total = src.shape[0]
    t2 = (u[src] - w[dst]).reshape(e_total // 2, 2 * D_H)
    w2t = W2.T
    w2bd = jnp.zeros((2 * D_H, 2 * D_H), jnp.float32)
    w2bd = w2bd.at[:D_H, :D_H].set(w2t).at[D_H:, D_H:].set(w2t)
    h2 = _edge_mlp2(t2, w2bd)
    selfpad = jnp.concatenate(
        [selfinit, jnp.zeros((_NPAD - N_NODES, D_H), jnp.float32)])
    agg = _scatter_max_sc(h2, dst, selfpad)[:N_NODES]
    return _global_mlp(agg, b2, G1.T, g1, G2.T, g2, G3.T, g3)


# speedup vs baseline: 3.3454x; 1.7626x over previous
import functools

import jax
import jax.numpy as jnp
from jax import lax
from jax.experimental import pallas as pl
from jax.experimental.pallas import tpu as pltpu
from jax.experimental.pallas import tpu_sc as plsc

_INTERPRET = False

N_NODES = 10000
D_X = 128
D_H = 64

_NW = 32
_BS = 320
_NPAD = _NW * _BS
_EW = 8000
_CH = 32


_NRING = 8
_SHIFT = 5
_UNROLL = 4


def _scatter_max_sc(h, dst, selfinit_pad):
    n_win = dst.shape[0] // _EW
    mesh = plsc.VectorSubcoreMesh(core_axis_name="c", subcore_axis_name="s",
                                  num_cores=2, num_subcores=16)

    @functools.partial(
        pl.kernel,
        out_type=jax.ShapeDtypeStruct((_NPAD, D_H), jnp.float32),
        mesh=mesh,
        compiler_params=pltpu.CompilerParams(needs_layout_passes=False),
        scratch_types=[
            pltpu.VMEM((_BS, D_H), jnp.float32),
            pltpu.VMEM((_EW,), jnp.int32),
            pltpu.VMEM((_EW // _CH + 2, _CH), jnp.int32),
            pltpu.VMEM((_EW + 192,), jnp.int32),
            [pltpu.VMEM((_CH, 2 * D_H), jnp.float32) for _ in range(_NRING)],
            [pltpu.SemaphoreType.DMA for _ in range(_NRING)],
        ],
    )
    def body(h_hbm, dst_hbm, self_hbm, out_hbm,
             agg, wdst, idbuf, nodbuf, rowbufs, sems):
        wid = lax.axis_index("s") * 2 + lax.axis_index("c")
        lo = wid * _BS
        pltpu.sync_copy(self_hbm.at[pl.ds(lo, _BS)], agg)
        iota = lax.iota(jnp.int32, 16)
        ftrue = iota < 16
        zeros16 = jnp.zeros((16,), jnp.int32)

        def window(win, _):
            wbase = win * _EW
            pltpu.sync_copy(dst_hbm.at[pl.ds(wbase, _EW)], wdst)

            def scan(v, cnt):
                for uu in range(_UNROLL):
                    vv = v * _UNROLL + uu
                    d16 = wdst[pl.ds(vv * 16, 16)]
                    rel = d16 - lo
                    m = (rel >= 0) & (rel < _BS)
                    ids = wbase + vv * 16 + iota
                    csum = jnp.cumsum(jnp.where(m, jnp.int32(1), jnp.int32(0)))
                    pos = cnt - 1 + csum
                    nodpar = rel | ((ids & 1) << 16)
                    plsc.store_scatter(idbuf, [pos >> _SHIFT,
                                               pos & (_CH - 1)],
                                       ids >> 1, mask=m)
                    plsc.store_scatter(nodbuf, [pos], nodpar, mask=m)
                    cnt = cnt + csum[15]
                return cnt

            cnt = lax.fori_loop(0, _EW // 16 // _UNROLL, scan, jnp.int32(0))

            def padz(k, _):
                pz = cnt + k * 16 + iota
                plsc.store_scatter(idbuf, [pz >> _SHIFT, pz & (_CH - 1)],
                                   zeros16, mask=ftrue)
                return 0
            lax.fori_loop(0, max(_CH // 16, 1), padz, 0)

            nch = (cnt + _CH - 1) >> _SHIFT

            for b in range(_NRING):
                @pl.when(b < nch)
                def _issue0(b=b):
                    pltpu.async_copy(h_hbm.at[idbuf.at[b]], rowbufs[b],
                                     sems[b])

            def group(g, _):
                for b in range(_NRING):
                    c = g * _NRING + b

                    @pl.when(c < nch)
                    def _do(b=b, c=c):
                        pltpu.make_async_copy(h_hbm.at[idbuf.at[c]],
                                              rowbufs[b], sems[b]).wait()
                        me = jnp.minimum(_CH, cnt - c * _CH)

                        def rmw(e, _):
                            nodpar = nodbuf[pl.ds(c * _CH + e, 16)][0]
                            node = nodpar & 0xFFFF
                            par = nodpar >> 16
                            for j in range(4):
                                sl = pl.ds(j * 16, 16)
                                hsl = pl.ds(par * D_H + j * 16, 16)
                                agg[node, sl] = jnp.maximum(
                                    agg[node, sl], rowbufs[b][e, hsl])
                            return 0
                        lax.fori_loop(0, me, rmw, 0)

                        @pl.when(c + _NRING < nch)
                        def _issue():
                            pltpu.async_copy(
                                h_hbm.at[idbuf.at[c + _NRING]],
                                rowbufs[b], sems[b])
                return 0

            lax.fori_loop(0, (nch + _NRING - 1) // _NRING, group, 0)
            return 0

        lax.fori_loop(0, n_win, window, 0)
        pltpu.sync_copy(agg, out_hbm.at[pl.ds(lo, _BS)])

    return body(h, dst, selfinit_pad)


_GC = 80


def _gather_sc(p, src, dst):
    e_total = src.shape[0]
    per_tile = e_total // _NW
    n_chunks = per_tile // _GC
    mesh = plsc.VectorSubcoreMesh(core_axis_name="c", subcore_axis_name="s",
                                  num_cores=2, num_subcores=16)

    @functools.partial(
        pl.kernel,
        out_type=jax.ShapeDtypeStruct((e_total // 2, 2 * D_H), jnp.float32),
        mesh=mesh,
        compiler_params=pltpu.CompilerParams(needs_layout_passes=False),
        scratch_types=[
            pltpu.VMEM_SHARED((N_NODES, 2 * D_H), jnp.float32),
            pltpu.VMEM((per_tile,), jnp.int32),
            pltpu.VMEM((per_tile,), jnp.int32),
            pltpu.VMEM((_GC, 2 * D_H), jnp.float32),
            pltpu.VMEM((_GC, 2 * D_H), jnp.float32),
            pltpu.VMEM((_GC // 2, 2 * D_H), jnp.float32),
            pltpu.SemaphoreType.DMA,
            pltpu.SemaphoreType.DMA,
        ],
    )
    def body(p_hbm, src_hbm, dst_hbm, t2_hbm,
             psp, srcw, dstw, abuf, bbuf, obuf, sema, semb):
        cid = lax.axis_index("c")
        sid = lax.axis_index("s")
        wid = sid * 2 + cid
        tb = wid * per_tile
        tb2 = wid * (per_tile // 2)

        @pl.when(sid == 0)
        def _stage():
            pltpu.sync_copy(p_hbm, psp)
        plsc.subcore_barrier()

        pltpu.sync_copy(src_hbm.at[pl.ds(tb, per_tile)], srcw)
        pltpu.sync_copy(dst_hbm.at[pl.ds(tb, per_tile)], dstw)

        def chunk(c, _):
            cb = c * _GC
            ca = pltpu.async_copy(psp.at[srcw.at[pl.ds(cb, _GC)]], abuf, sema)
            cbm = pltpu.async_copy(psp.at[dstw.at[pl.ds(cb, _GC)]], bbuf, semb)
            ca.wait()
            cbm.wait()

            def pair(i, _):
                for j in range(4):
                    slo = pl.ds(j * 16, 16)
                    shi = pl.ds(D_H + j * 16, 16)
                    obuf[i, slo] = abuf[2 * i, slo] - bbuf[2 * i, shi]
                    obuf[i, shi] = abuf[2 * i + 1, slo] - bbuf[2 * i + 1, shi]
                return 0
            lax.fori_loop(0, _GC // 2, pair, 0)
            pltpu.sync_copy(
                obuf, t2_hbm.at[pl.ds(tb2 + c * (_GC // 2), _GC // 2)])
            return 0

        lax.fori_loop(0, n_chunks, chunk, 0)

    return body(p, src, dst)


def _node_pre_body(x_ref, pos_ref, w1xt_ref, w1pt_ref, b1_ref, w2t_ref,
                   u_ref, w_ref, self_ref):
    xb = x_ref[...]
    pb = pos_ref[...]
    w_blk = jnp.dot(pb, w1pt_ref[...], preferred_element_type=jnp.float32)
    ux = jnp.dot(xb, w1xt_ref[...], preferred_element_type=jnp.float32)
    u_blk = ux + w_blk + b1_ref[...]
    u_ref[...] = u_blk
    w_ref[...] = w_blk
    self_ref[...] = jnp.dot(jax.nn.relu(ux + b1_ref[...]), w2t_ref[...],
                            preferred_element_type=jnp.float32)


def _node_pre(x, pos, w1xt, w1pt, b1, w2t, bn=1000):
    n = x.shape[0]
    grid = (n // bn,)
    return pl.pallas_call(
        _node_pre_body,
        grid=grid,
        in_specs=[
            pl.BlockSpec((bn, D_X), lambda i: (i, 0)),
            pl.BlockSpec((bn, 3), lambda i: (i, 0)),
            pl.BlockSpec((D_X, D_H), lambda i: (0, 0)),
            pl.BlockSpec((3, D_H), lambda i: (0, 0)),
            pl.BlockSpec((1, D_H), lambda i: (0, 0)),
            pl.BlockSpec((D_H, D_H), lambda i: (0, 0)),
        ],
        out_specs=[
            pl.BlockSpec((bn, D_H), lambda i: (i, 0)),
            pl.BlockSpec((bn, D_H), lambda i: (i, 0)),
            pl.BlockSpec((bn, D_H), lambda i: (i, 0)),
        ],
        out_shape=[
            jax.ShapeDtypeStruct((n, D_H), jnp.float32),
            jax.ShapeDtypeStruct((n, D_H), jnp.float32),
            jax.ShapeDtypeStruct((n, D_H), jnp.float32),
        ],
        interpret=_INTERPRET,
    )(x, pos, w1xt, w1pt, b1.reshape(1, D_H), w2t)


def _edge_mlp_body(t_ref, w2t_ref, h_ref):
    h_ref[...] = jnp.dot(jax.nn.relu(t_ref[...]), w2t_ref[...],
                         preferred_element_type=jnp.float32)


def _edge_mlp2(t2, w2bd, be=1000):
    e2 = t2.shape[0]
    grid = (e2 // be,)
    return pl.pallas_call(
        _edge_mlp_body,
        grid=grid,
        in_specs=[
            pl.BlockSpec((be, 2 * D_H), lambda i: (i, 0)),
            pl.BlockSpec((2 * D_H, 2 * D_H), lambda i: (0, 0)),
        ],
        out_specs=pl.BlockSpec((be, 2 * D_H), lambda i: (i, 0)),
        out_shape=jax.ShapeDtypeStruct((e2, 2 * D_H), jnp.float32),
        interpret=_INTERPRET,
    )(t2, w2bd)


def _global_mlp_body(a_ref, b2_ref, g1t_ref, g1_ref, g2t_ref, g2_ref,
                     g3t_ref, g3_ref, o_ref):
    a = a_ref[...] + b2_ref[...]
    a = jax.nn.relu(jnp.dot(a, g1t_ref[...], preferred_element_type=jnp.float32)
                    + g1_ref[...])
    a = jax.nn.relu(jnp.dot(a, g2t_ref[...], preferred_element_type=jnp.float32)
                    + g2_ref[...])
    o_ref[...] = jnp.dot(a, g3t_ref[...], preferred_element_type=jnp.float32) \
        + g3_ref[...]


def _global_mlp(agg, b2, g1t, g1, g2t, g2, g3t, g3, bn=1000):
    n = agg.shape[0]
    grid = (n // bn,)
    return pl.pallas_call(
        _global_mlp_body,
        grid=grid,
        in_specs=[
            pl.BlockSpec((bn, D_H), lambda i: (i, 0)),
            pl.BlockSpec((1, D_H), lambda i: (0, 0)),
            pl.BlockSpec((D_H, 32), lambda i: (0, 0)),
            pl.BlockSpec((1, 32), lambda i: (0, 0)),
            pl.BlockSpec((32, 128), lambda i: (0, 0)),
            pl.BlockSpec((1, 128), lambda i: (0, 0)),
            pl.BlockSpec((128, 128), lambda i: (0, 0)),
            pl.BlockSpec((1, 128), lambda i: (0, 0)),
        ],
        out_specs=pl.BlockSpec((bn, 128), lambda i: (i, 0)),
        out_shape=jax.ShapeDtypeStruct((n, 128), jnp.float32),
        interpret=_INTERPRET,
    )(agg, b2.reshape(1, D_H), g1t, g1.reshape(1, 32), g2t, g2.reshape(1, 128),
      g3t, g3.reshape(1, 128))


def kernel(x, pos, edge_index, W1, b1, W2, b2, G1, g1, G2, g2, G3, g3):
    src = edge_index[0].astype(jnp.int32)
    dst = edge_index[1].astype(jnp.int32)
    w1xt = W1[:, :D_X].T
    w1pt = W1[:, D_X:].T
    u, w, selfinit = _node_pre(x, pos, w1xt, w1pt, b1, W2.T)
    p = jnp.concatenate([u, w], axis=1)
    t2 = _gather_sc(p, src, dst)
    w2t = W2.T
    w2bd = jnp.zeros((2 * D_H, 2 * D_H), jnp.float32)
    w2bd = w2bd.at[:D_H, :D_H].set(w2t).at[D_H:, D_H:].set(w2t)
    h2 = _edge_mlp2(t2, w2bd)
    selfpad = jnp.concatenate(
        [selfinit, jnp.zeros((_NPAD - N_NODES, D_H), jnp.float32)])
    agg = _scatter_max_sc(h2, dst, selfpad)[:N_NODES]
    return _global_mlp(agg, b2, G1.T, g1, G2.T, g2, G3.T, g3)


# speedup vs baseline: 3.7694x; 1.1267x over previous
import functools

import jax
import jax.numpy as jnp
from jax import lax
from jax.experimental import pallas as pl
from jax.experimental.pallas import tpu as pltpu
from jax.experimental.pallas import tpu_sc as plsc

_INTERPRET = False

N_NODES = 10000
D_X = 128
D_H = 64

_NW = 32
_BS = 320
_NPAD = _NW * _BS
_EW = 8000
_CH = 32


_NRING = 8
_SHIFT = 5
_UNROLL = 4


def _scatter_max_sc(h, dst, selfinit_pad):
    n_win = dst.shape[0] // _EW
    mesh = plsc.VectorSubcoreMesh(core_axis_name="c", subcore_axis_name="s",
                                  num_cores=2, num_subcores=16)

    @functools.partial(
        pl.kernel,
        out_type=jax.ShapeDtypeStruct((_NPAD, D_H), jnp.float32),
        mesh=mesh,
        compiler_params=pltpu.CompilerParams(needs_layout_passes=False),
        scratch_types=[
            pltpu.VMEM((_BS, D_H), jnp.float32),
            pltpu.VMEM((_EW,), jnp.int32),
            pltpu.VMEM((_EW // _CH + 2, _CH), jnp.int32),
            pltpu.VMEM((_EW + 192,), jnp.int32),
            [pltpu.VMEM((_CH, 2 * D_H), jnp.float32) for _ in range(_NRING)],
            [pltpu.SemaphoreType.DMA for _ in range(_NRING)],
        ],
    )
    def body(h_hbm, dst_hbm, self_hbm, out_hbm,
             agg, wdst, idbuf, nodbuf, rowbufs, sems):
        wid = lax.axis_index("s") * 2 + lax.axis_index("c")
        lo = wid * _BS
        pltpu.sync_copy(self_hbm.at[pl.ds(lo, _BS)], agg)
        iota = lax.iota(jnp.int32, 16)
        ftrue = iota < 16
        zeros16 = jnp.zeros((16,), jnp.int32)

        def window(win, _):
            wbase = win * _EW
            pltpu.sync_copy(dst_hbm.at[pl.ds(wbase, _EW)], wdst)

            def scan(v, cnt):
                for uu in range(_UNROLL):
                    vv = v * _UNROLL + uu
                    d16 = wdst[pl.ds(vv * 16, 16)]
                    rel = d16 - lo
                    m = (rel >= 0) & (rel < _BS)
                    ids = wbase + vv * 16 + iota
                    csum = jnp.cumsum(jnp.where(m, jnp.int32(1), jnp.int32(0)))
                    pos = cnt - 1 + csum
                    nodpar = rel | ((ids & 1) << 16)
                    plsc.store_scatter(idbuf, [pos >> _SHIFT,
                                               pos & (_CH - 1)],
                                       ids >> 1, mask=m)
                    plsc.store_scatter(nodbuf, [pos], nodpar, mask=m)
                    cnt = cnt + csum[15]
                return cnt

            cnt = lax.fori_loop(0, _EW // 16 // _UNROLL, scan, jnp.int32(0))

            def padz(k, _):
                pz = cnt + k * 16 + iota
                plsc.store_scatter(idbuf, [pz >> _SHIFT, pz & (_CH - 1)],
                                   wid * 128 + iota, mask=ftrue)
                return 0
            lax.fori_loop(0, max(_CH // 16, 1), padz, 0)

            nch = (cnt + _CH - 1) >> _SHIFT

            for b in range(_NRING):
                @pl.when(b < nch)
                def _issue0(b=b):
                    pltpu.async_copy(h_hbm.at[idbuf.at[b]], rowbufs[b],
                                     sems[b])

            def group(g, _):
                for b in range(_NRING):
                    c = g * _NRING + b

                    @pl.when(c < nch)
                    def _do(b=b, c=c):
                        pltpu.make_async_copy(h_hbm.at[idbuf.at[c]],
                                              rowbufs[b], sems[b]).wait()
                        me = jnp.minimum(_CH, cnt - c * _CH)

                        def rmw(e, _):
                            nodpar = nodbuf[pl.ds(c * _CH + e, 16)][0]
                            node = nodpar & 0xFFFF
                            par = nodpar >> 16
                            for j in range(4):
                                sl = pl.ds(j * 16, 16)
                                hsl = pl.ds(par * D_H + j * 16, 16)
                                agg[node, sl] = jnp.maximum(
                                    agg[node, sl], rowbufs[b][e, hsl])
                            return 0
                        lax.fori_loop(0, me, rmw, 0)

                        @pl.when(c + _NRING < nch)
                        def _issue():
                            pltpu.async_copy(
                                h_hbm.at[idbuf.at[c + _NRING]],
                                rowbufs[b], sems[b])
                return 0

            lax.fori_loop(0, (nch + _NRING - 1) // _NRING, group, 0)
            return 0

        lax.fori_loop(0, n_win, window, 0)
        pltpu.sync_copy(agg, out_hbm.at[pl.ds(lo, _BS)])

    return body(h, dst, selfinit_pad)


_GC = 80


def _gather_sc(p, src, dst):
    e_total = src.shape[0]
    per_tile = e_total // _NW
    n_chunks = per_tile // _GC
    mesh = plsc.VectorSubcoreMesh(core_axis_name="c", subcore_axis_name="s",
                                  num_cores=2, num_subcores=16)

    @functools.partial(
        pl.kernel,
        out_type=jax.ShapeDtypeStruct((e_total // 2, 2 * D_H), jnp.float32),
        mesh=mesh,
        compiler_params=pltpu.CompilerParams(needs_layout_passes=False),
        scratch_types=[
            pltpu.VMEM_SHARED((N_NODES, 2 * D_H), jnp.float32),
            pltpu.VMEM((per_tile,), jnp.int32),
            pltpu.VMEM((per_tile,), jnp.int32),
            pltpu.VMEM((_GC, 2 * D_H), jnp.float32),
            pltpu.VMEM((_GC, 2 * D_H), jnp.float32),
            pltpu.VMEM((_GC // 2, 2 * D_H), jnp.float32),
            pltpu.SemaphoreType.DMA,
            pltpu.SemaphoreType.DMA,
        ],
    )
    def body(p_hbm, src_hbm, dst_hbm, t2_hbm,
             psp, srcw, dstw, abuf, bbuf, obuf, sema, semb):
        cid = lax.axis_index("c")
        sid = lax.axis_index("s")
        wid = sid * 2 + cid
        tb = wid * per_tile
        tb2 = wid * (per_tile // 2)

        @pl.when(sid == 0)
        def _stage():
            pltpu.sync_copy(p_hbm, psp)
        plsc.subcore_barrier()

        pltpu.sync_copy(src_hbm.at[pl.ds(tb, per_tile)], srcw)
        pltpu.sync_copy(dst_hbm.at[pl.ds(tb, per_tile)], dstw)

        def chunk(c, _):
            cb = c * _GC
            ca = pltpu.async_copy(psp.at[srcw.at[pl.ds(cb, _GC)]], abuf, sema)
            cbm = pltpu.async_copy(psp.at[dstw.at[pl.ds(cb, _GC)]], bbuf, semb)
            ca.wait()
            cbm.wait()

            def pair(i, _):
                for j in range(4):
                    slo = pl.ds(j * 16, 16)
                    shi = pl.ds(D_H + j * 16, 16)
                    obuf[i, slo] = abuf[2 * i, slo] - bbuf[2 * i, shi]
                    obuf[i, shi] = abuf[2 * i + 1, slo] - bbuf[2 * i + 1, shi]
                return 0
            lax.fori_loop(0, _GC // 2, pair, 0)
            pltpu.sync_copy(
                obuf, t2_hbm.at[pl.ds(tb2 + c * (_GC // 2), _GC // 2)])
            return 0

        lax.fori_loop(0, n_chunks, chunk, 0)

    return body(p, src, dst)


def _node_pre_body(x_ref, pos_ref, w1xt_ref, w1pt_ref, b1_ref, w2t_ref,
                   u_ref, w_ref, self_ref):
    xb = x_ref[...]
    pb = pos_ref[...]
    w_blk = jnp.dot(pb, w1pt_ref[...], preferred_element_type=jnp.float32)
    ux = jnp.dot(xb, w1xt_ref[...], preferred_element_type=jnp.float32)
    u_blk = ux + w_blk + b1_ref[...]
    u_ref[...] = u_blk
    w_ref[...] = w_blk
    self_ref[...] = jnp.dot(jax.nn.relu(ux + b1_ref[...]), w2t_ref[...],
                            preferred_element_type=jnp.float32)


def _node_pre(x, pos, w1xt, w1pt, b1, w2t, bn=1000):
    n = x.shape[0]
    grid = (n // bn,)
    return pl.pallas_call(
        _node_pre_body,
        grid=grid,
        in_specs=[
            pl.BlockSpec((bn, D_X), lambda i: (i, 0)),
            pl.BlockSpec((bn, 3), lambda i: (i, 0)),
            pl.BlockSpec((D_X, D_H), lambda i: (0, 0)),
            pl.BlockSpec((3, D_H), lambda i: (0, 0)),
            pl.BlockSpec((1, D_H), lambda i: (0, 0)),
            pl.BlockSpec((D_H, D_H), lambda i: (0, 0)),
        ],
        out_specs=[
            pl.BlockSpec((bn, D_H), lambda i: (i, 0)),
            pl.BlockSpec((bn, D_H), lambda i: (i, 0)),
            pl.BlockSpec((bn, D_H), lambda i: (i, 0)),
        ],
        out_shape=[
            jax.ShapeDtypeStruct((n, D_H), jnp.float32),
            jax.ShapeDtypeStruct((n, D_H), jnp.float32),
            jax.ShapeDtypeStruct((n, D_H), jnp.float32),
        ],
        interpret=_INTERPRET,
    )(x, pos, w1xt, w1pt, b1.reshape(1, D_H), w2t)


def _edge_mlp_body(t_ref, w2t_ref, h_ref):
    h_ref[...] = jnp.dot(jax.nn.relu(t_ref[...]), w2t_ref[...],
                         preferred_element_type=jnp.float32)


def _edge_mlp2(t2, w2bd, be=1000):
    e2 = t2.shape[0]
    grid = (e2 // be,)
    return pl.pallas_call(
        _edge_mlp_body,
        grid=grid,
        in_specs=[
            pl.BlockSpec((be, 2 * D_H), lambda i: (i, 0)),
            pl.BlockSpec((2 * D_H, 2 * D_H), lambda i: (0, 0)),
        ],
        out_specs=pl.BlockSpec((be, 2 * D_H), lambda i: (i, 0)),
        out_shape=jax.ShapeDtypeStruct((e2, 2 * D_H), jnp.float32),
        interpret=_INTERPRET,
    )(t2, w2bd)


def _global_mlp_body(a_ref, b2_ref, g1t_ref, g1_ref, g2t_ref, g2_ref,
                     g3t_ref, g3_ref, o_ref):
    a = a_ref[...] + b2_ref[...]
    a = jax.nn.relu(jnp.dot(a, g1t_ref[...], preferred_element_type=jnp.float32)
                    + g1_ref[...])
    a = jax.nn.relu(jnp.dot(a, g2t_ref[...], preferred_element_type=jnp.float32)
                    + g2_ref[...])
    o_ref[...] = jnp.dot(a, g3t_ref[...], preferred_element_type=jnp.float32) \
        + g3_ref[...]


def _global_mlp(agg, b2, g1t, g1, g2t, g2, g3t, g3, bn=1000):
    n = agg.shape[0]
    grid = (n // bn,)
    return pl.pallas_call(
        _global_mlp_body,
        grid=grid,
        in_specs=[
            pl.BlockSpec((bn, D_H), lambda i: (i, 0)),
            pl.BlockSpec((1, D_H), lambda i: (0, 0)),
            pl.BlockSpec((D_H, 32), lambda i: (0, 0)),
            pl.BlockSpec((1, 32), lambda i: (0, 0)),
            pl.BlockSpec((32, 128), lambda i: (0, 0)),
            pl.BlockSpec((1, 128), lambda i: (0, 0)),
            pl.BlockSpec((128, 128), lambda i: (0, 0)),
            pl.BlockSpec((1, 128), lambda i: (0, 0)),
        ],
        out_specs=pl.BlockSpec((bn, 128), lambda i: (i, 0)),
        out_shape=jax.ShapeDtypeStruct((n, 128), jnp.float32),
        interpret=_INTERPRET,
    )(agg, b2.reshape(1, D_H), g1t, g1.reshape(1, 32), g2t, g2.reshape(1, 128),
      g3t, g3.reshape(1, 128))


def kernel(x, pos, edge_index, W1, b1, W2, b2, G1, g1, G2, g2, G3, g3):
    src = edge_index[0].astype(jnp.int32)
    dst = edge_index[1].astype(jnp.int32)
    w1xt = W1[:, :D_X].T
    w1pt = W1[:, D_X:].T
    u, w, selfinit = _node_pre(x, pos, w1xt, w1pt, b1, W2.T)
    p = jnp.concatenate([u, w], axis=1)
    t2 = _gather_sc(p, src, dst)
    w2t = W2.T
    w2bd = jnp.zeros((2 * D_H, 2 * D_H), jnp.float32)
    w2bd = w2bd.at[:D_H, :D_H].set(w2t).at[D_H:, D_H:].set(w2t)
    h2 = _edge_mlp2(t2, w2bd)
    selfpad = jnp.concatenate(
        [selfinit, jnp.zeros((_NPAD - N_NODES, D_H), jnp.float32)])
    agg = _scatter_max_sc(h2, dst, selfpad)[:N_NODES]
    return _global_mlp(agg, b2, G1.T, g1, G2.T, g2, G3.T, g3)


# speedup vs baseline: 3.7702x; 1.0002x over previous
import functools

import jax
import jax.numpy as jnp
from jax import lax
from jax.experimental import pallas as pl
from jax.experimental.pallas import tpu as pltpu
from jax.experimental.pallas import tpu_sc as plsc

N_NODES = 10000
D_X = 128
D_H = 64

_NW = 32
_BS = 320
_NPAD = _NW * _BS
_EW = 8000
_CH = 32


_NRING = 8
_SHIFT = 5
_UNROLL = 4


def _scatter_max_sc(h, dst, selfinit_pad):
    n_win = dst.shape[0] // _EW
    mesh = plsc.VectorSubcoreMesh(core_axis_name="c", subcore_axis_name="s",
                                  num_cores=2, num_subcores=16)

    @functools.partial(
        pl.kernel,
        out_type=jax.ShapeDtypeStruct((_NPAD, D_H), jnp.float32),
        mesh=mesh,
        compiler_params=pltpu.CompilerParams(needs_layout_passes=False),
        scratch_types=[
            pltpu.VMEM((_BS, D_H), jnp.float32),
            pltpu.VMEM((_EW,), jnp.int32),
            pltpu.VMEM((_EW // _CH + 2, _CH), jnp.int32),
            pltpu.VMEM((_EW + 192,), jnp.int32),
            [pltpu.VMEM((_CH, 2 * D_H), jnp.float32) for _ in range(_NRING)],
            [pltpu.SemaphoreType.DMA for _ in range(_NRING)],
        ],
    )
    def body(h_hbm, dst_hbm, self_hbm, out_hbm,
             agg, wdst, idbuf, nodbuf, rowbufs, sems):
        wid = lax.axis_index("s") * 2 + lax.axis_index("c")
        lo = wid * _BS
        pltpu.sync_copy(self_hbm.at[pl.ds(lo, _BS)], agg)
        iota = lax.iota(jnp.int32, 16)
        ftrue = iota < 16

        def window(win, _):
            wbase = win * _EW
            pltpu.sync_copy(dst_hbm.at[pl.ds(wbase, _EW)], wdst)

            def scan(v, cnt):
                for uu in range(_UNROLL):
                    vv = v * _UNROLL + uu
                    d16 = wdst[pl.ds(vv * 16, 16)]
                    rel = d16 - lo
                    m = (rel >= 0) & (rel < _BS)
                    ids = wbase + vv * 16 + iota
                    csum = jnp.cumsum(jnp.where(m, jnp.int32(1), jnp.int32(0)))
                    pos = cnt - 1 + csum
                    nodpar = rel | ((ids & 1) << 16)
                    plsc.store_scatter(idbuf, [pos >> _SHIFT,
                                               pos & (_CH - 1)],
                                       ids >> 1, mask=m)
                    plsc.store_scatter(nodbuf, [pos], nodpar, mask=m)
                    cnt = cnt + csum[15]
                return cnt

            cnt = lax.fori_loop(0, _EW // 16 // _UNROLL, scan, jnp.int32(0))

            def padz(k, _):
                pz = cnt + k * 16 + iota
                plsc.store_scatter(idbuf, [pz >> _SHIFT, pz & (_CH - 1)],
                                   wid * 128 + iota, mask=ftrue)
                return 0
            lax.fori_loop(0, max(_CH // 16, 1), padz, 0)

            nch = (cnt + _CH - 1) >> _SHIFT

            for b in range(_NRING):
                @pl.when(b < nch)
                def _issue0(b=b):
                    pltpu.async_copy(h_hbm.at[idbuf.at[b]], rowbufs[b],
                                     sems[b])

            def group(g, _):
                for b in range(_NRING):
                    c = g * _NRING + b

                    @pl.when(c < nch)
                    def _do(b=b, c=c):
                        pltpu.make_async_copy(h_hbm.at[idbuf.at[c]],
                                              rowbufs[b], sems[b]).wait()
                        me = jnp.minimum(_CH, cnt - c * _CH)

                        def rmw(e, _):
                            nodpar = nodbuf[pl.ds(c * _CH + e, 16)][0]
                            node = nodpar & 0xFFFF
                            par = nodpar >> 16
                            for j in range(4):
                                sl = pl.ds(j * 16, 16)
                                hsl = pl.ds(par * D_H + j * 16, 16)
                                agg[node, sl] = jnp.maximum(
                                    agg[node, sl], rowbufs[b][e, hsl])
                            return 0
                        lax.fori_loop(0, me, rmw, 0)

                        @pl.when(c + _NRING < nch)
                        def _issue():
                            pltpu.async_copy(
                                h_hbm.at[idbuf.at[c + _NRING]],
                                rowbufs[b], sems[b])
                return 0

            lax.fori_loop(0, (nch + _NRING - 1) // _NRING, group, 0)
            return 0

        lax.fori_loop(0, n_win, window, 0)
        pltpu.sync_copy(agg, out_hbm.at[pl.ds(lo, _BS)])

    return body(h, dst, selfinit_pad)


_GC = 80


def _gather_sc(p, src, dst):
    e_total = src.shape[0]
    per_tile = e_total // _NW
    n_chunks = per_tile // _GC
    mesh = plsc.VectorSubcoreMesh(core_axis_name="c", subcore_axis_name="s",
                                  num_cores=2, num_subcores=16)

    @functools.partial(
        pl.kernel,
        out_type=jax.ShapeDtypeStruct((e_total // 2, 2 * D_H), jnp.float32),
        mesh=mesh,
        compiler_params=pltpu.CompilerParams(needs_layout_passes=False),
        scratch_types=[
            pltpu.VMEM_SHARED((N_NODES, 2 * D_H), jnp.float32),
            pltpu.VMEM((per_tile,), jnp.int32),
            pltpu.VMEM((per_tile,), jnp.int32),
            pltpu.VMEM((_GC, 2 * D_H), jnp.float32),
            pltpu.VMEM((_GC, 2 * D_H), jnp.float32),
            pltpu.VMEM((_GC // 2, 2 * D_H), jnp.float32),
            pltpu.SemaphoreType.DMA,
            pltpu.SemaphoreType.DMA,
        ],
    )
    def body(p_hbm, src_hbm, dst_hbm, t2_hbm,
             psp, srcw, dstw, abuf, bbuf, obuf, sema, semb):
        cid = lax.axis_index("c")
        sid = lax.axis_index("s")
        wid = sid * 2 + cid
        tb = wid * per_tile
        tb2 = wid * (per_tile // 2)

        @pl.when(sid == 0)
        def _stage():
            pltpu.sync_copy(p_hbm, psp)
        plsc.subcore_barrier()

        pltpu.sync_copy(src_hbm.at[pl.ds(tb, per_tile)], srcw)
        pltpu.sync_copy(dst_hbm.at[pl.ds(tb, per_tile)], dstw)

        def chunk(c, _):
            cb = c * _GC
            ca = pltpu.async_copy(psp.at[srcw.at[pl.ds(cb, _GC)]], abuf, sema)
            cbm = pltpu.async_copy(psp.at[dstw.at[pl.ds(cb, _GC)]], bbuf, semb)
            ca.wait()
            cbm.wait()

            def pair(i, _):
                for j in range(4):
                    slo = pl.ds(j * 16, 16)
                    shi = pl.ds(D_H + j * 16, 16)
                    obuf[i, slo] = abuf[2 * i, slo] - bbuf[2 * i, shi]
                    obuf[i, shi] = abuf[2 * i + 1, slo] - bbuf[2 * i + 1, shi]
                return 0
            lax.fori_loop(0, _GC // 2, pair, 0)
            pltpu.sync_copy(
                obuf, t2_hbm.at[pl.ds(tb2 + c * (_GC // 2), _GC // 2)])
            return 0

        lax.fori_loop(0, n_chunks, chunk, 0)

    return body(p, src, dst)


def _node_pre_body(x_ref, pos_ref, w1xt_ref, w1pt_ref, b1_ref, w2t_ref,
                   u_ref, w_ref, self_ref):
    xb = x_ref[...]
    pb = pos_ref[...]
    w_blk = jnp.dot(pb, w1pt_ref[...], preferred_element_type=jnp.float32)
    ux = jnp.dot(xb, w1xt_ref[...], preferred_element_type=jnp.float32)
    u_blk = ux + w_blk + b1_ref[...]
    u_ref[...] = u_blk
    w_ref[...] = w_blk
    self_ref[...] = jnp.dot(jax.nn.relu(ux + b1_ref[...]), w2t_ref[...],
                            preferred_element_type=jnp.float32)


def _node_pre(x, pos, w1xt, w1pt, b1, w2t, bn=1000):
    n = x.shape[0]
    grid = (n // bn,)
    return pl.pallas_call(
        _node_pre_body,
        grid=grid,
        in_specs=[
            pl.BlockSpec((bn, D_X), lambda i: (i, 0)),
            pl.BlockSpec((bn, 3), lambda i: (i, 0)),
            pl.BlockSpec((D_X, D_H), lambda i: (0, 0)),
            pl.BlockSpec((3, D_H), lambda i: (0, 0)),
            pl.BlockSpec((1, D_H), lambda i: (0, 0)),
            pl.BlockSpec((D_H, D_H), lambda i: (0, 0)),
        ],
        out_specs=[
            pl.BlockSpec((bn, D_H), lambda i: (i, 0)),
            pl.BlockSpec((bn, D_H), lambda i: (i, 0)),
            pl.BlockSpec((bn, D_H), lambda i: (i, 0)),
        ],
        out_shape=[
            jax.ShapeDtypeStruct((n, D_H), jnp.float32),
            jax.ShapeDtypeStruct((n, D_H), jnp.float32),
            jax.ShapeDtypeStruct((n, D_H), jnp.float32),
        ],
    )(x, pos, w1xt, w1pt, b1.reshape(1, D_H), w2t)


def _edge_mlp_body(t_ref, w2t_ref, h_ref):
    h_ref[...] = jnp.dot(jax.nn.relu(t_ref[...]), w2t_ref[...],
                         preferred_element_type=jnp.float32)


def _edge_mlp2(t2, w2bd, be=1000):
    e2 = t2.shape[0]
    grid = (e2 // be,)
    return pl.pallas_call(
        _edge_mlp_body,
        grid=grid,
        in_specs=[
            pl.BlockSpec((be, 2 * D_H), lambda i: (i, 0)),
            pl.BlockSpec((2 * D_H, 2 * D_H), lambda i: (0, 0)),
        ],
        out_specs=pl.BlockSpec((be, 2 * D_H), lambda i: (i, 0)),
        out_shape=jax.ShapeDtypeStruct((e2, 2 * D_H), jnp.float32),
    )(t2, w2bd)


def _global_mlp_body(a_ref, b2_ref, g1t_ref, g1_ref, g2t_ref, g2_ref,
                     g3t_ref, g3_ref, o_ref):
    a = a_ref[...] + b2_ref[...]
    a = jax.nn.relu(jnp.dot(a, g1t_ref[...], preferred_element_type=jnp.float32)
                    + g1_ref[...])
    a = jax.nn.relu(jnp.dot(a, g2t_ref[...], preferred_element_type=jnp.float32)
                    + g2_ref[...])
    o_ref[...] = jnp.dot(a, g3t_ref[...], preferred_element_type=jnp.float32) \
        + g3_ref[...]


def _global_mlp(agg, b2, g1t, g1, g2t, g2, g3t, g3, bn=1000):
    n = agg.shape[0]
    grid = (n // bn,)
    return pl.pallas_call(
        _global_mlp_body,
        grid=grid,
        in_specs=[
            pl.BlockSpec((bn, D_H), lambda i: (i, 0)),
            pl.BlockSpec((1, D_H), lambda i: (0, 0)),
            pl.BlockSpec((D_H, 32), lambda i: (0, 0)),
            pl.BlockSpec((1, 32), lambda i: (0, 0)),
            pl.BlockSpec((32, 128), lambda i: (0, 0)),
            pl.BlockSpec((1, 128), lambda i: (0, 0)),
            pl.BlockSpec((128, 128), lambda i: (0, 0)),
            pl.BlockSpec((1, 128), lambda i: (0, 0)),
        ],
        out_specs=pl.BlockSpec((bn, 128), lambda i: (i, 0)),
        out_shape=jax.ShapeDtypeStruct((n, 128), jnp.float32),
    )(agg, b2.reshape(1, D_H), g1t, g1.reshape(1, 32), g2t, g2.reshape(1, 128),
      g3t, g3.reshape(1, 128))


def kernel(x, pos, edge_index, W1, b1, W2, b2, G1, g1, G2, g2, G3, g3):
    src = edge_index[0].astype(jnp.int32)
    dst = edge_index[1].astype(jnp.int32)
    w1xt = W1[:, :D_X].T
    w1pt = W1[:, D_X:].T
    u, w, selfinit = _node_pre(x, pos, w1xt, w1pt, b1, W2.T)
    p = jnp.concatenate([u, w], axis=1)
    t2 = _gather_sc(p, src, dst)
    w2t = W2.T
    w2bd = jnp.zeros((2 * D_H, 2 * D_H), jnp.float32)
    w2bd = w2bd.at[:D_H, :D_H].set(w2t).at[D_H:, D_H:].set(w2t)
    h2 = _edge_mlp2(t2, w2bd)
    selfpad = jnp.concatenate(
        [selfinit, jnp.zeros((_NPAD - N_NODES, D_H), jnp.float32)])
    agg = _scatter_max_sc(h2, dst, selfpad)[:N_NODES]
    return _global_mlp(agg, b2, G1.T, g1, G2.T, g2, G3.T, g3)
